# baseline (speedup 1.0000x reference)
import functools

import jax
import jax.numpy as jnp
from jax import lax
from jax.experimental import pallas as pl
from jax.experimental.pallas import tpu as pltpu
from jax.experimental.pallas import tpu_sc as plsc

N = 10000
E = 320000
DIN = 128
DH = DIN // 2
DE = 16
DOUT = 128

NC = 2
NS = 16
NW = NC * NS
NACC = 10240
RPT_A = NACC // NS

EPS = E // NS
CHA = 80
NCHA = EPS // CHA
NGRP = NCHA // 5

EPT = E // NW
CHC = 50
NCHC = EPT // CHC
NPS = N // NS

_mesh = plsc.VectorSubcoreMesh(
    core_axis_name="c", subcore_axis_name="s", num_cores=NC, num_subcores=NS)
_sc_params = pltpu.CompilerParams(
    use_tc_tiling_on_sc=False, needs_layout_passes=False)


@functools.partial(
    pl.kernel,
    out_type=(
        jax.ShapeDtypeStruct((NC, NACC, DH), jnp.float32),
        jax.ShapeDtypeStruct((NACC, DE), jnp.float32),
        jax.ShapeDtypeStruct((NS, N), jnp.float32),
    ),
    mesh=_mesh,
    scratch_types=[
        [pltpu.VMEM((5, CHA), jnp.int32) for _ in range(2)],
        pltpu.VMEM((NCHA, CHA), jnp.int32),
        [pltpu.VMEM((CHA, DH), jnp.float32) for _ in range(5)],
        [pltpu.VMEM((CHA, DE), jnp.float32) for _ in range(5)],
        pltpu.VMEM((N,), jnp.float32),
        pltpu.VMEM_SHARED((NACC, DH), jnp.float32),
        pltpu.VMEM_SHARED((NACC, DE), jnp.float32),
        [pltpu.SemaphoreType.DMA for _ in range(2)],
        [pltpu.SemaphoreType.DMA for _ in range(5)],
        [pltpu.SemaphoreType.DMA for _ in range(5)],
        [pltpu.SemaphoreType.DMA for _ in range(5)],
        [pltpu.SemaphoreType.DMA for _ in range(5)],
    ],
    compiler_params=_sc_params,
)
def _scatter_kernel(nf_view, efeats2, src_r, dst_r, zn, ze,
                    out_n, out_e, out_deg,
                    isr, idx_d, rows, erows, deg_l,
                    acc_n, acc_e, isl, gs, ss, el, es):
    cid = lax.axis_index("c")
    sid = lax.axis_index("s")
    on_c0 = cid == 0
    on_c1 = cid == 1
    cvec = jnp.zeros((16,), jnp.int32) + cid
    two = jnp.full((16,), 2, jnp.int32)

    r0 = sid * RPT_A
    pltpu.sync_copy(zn.at[pl.ds(r0, RPT_A)], acc_n.at[pl.ds(r0, RPT_A)])

    @pl.when(on_c0)
    def _():
        pltpu.sync_copy(ze.at[pl.ds(r0, RPT_A)],
                        acc_e.at[pl.ds(r0, RPT_A)])

    pltpu.sync_copy(dst_r.at[sid], idx_d)

    ebase = sid * EPS

    def isl_start(g, e):
        pltpu.async_copy(src_r.at[sid].at[pl.ds(g * 5, 5)], isr[e], isl[e])

    def isl_wait(e):
        pltpu.make_async_copy(src_r.at[sid].at[pl.ds(0, 5)],
                              isr[e], isl[e]).wait()
        for r in range(5):
            for k in range(CHA // 16):
                sl = pl.ds(k * 16, 16)
                isr[e][r, sl] = isr[e][r, sl] * two + cvec

    def g_start(row_ref, b):
        pltpu.async_copy(nf_view.at[row_ref], rows[b], gs[b])

    def g_wait(b):
        pltpu.make_async_copy(nf_view.at[isr[0].at[0]], rows[b],
                              gs[b]).wait()

    def e_start(j, b):
        pltpu.async_copy(efeats2.at[pl.ds(ebase + j * CHA, CHA)],
                         erows[b], el[b])

    def e_wait(b):
        pltpu.make_async_copy(efeats2.at[pl.ds(0, CHA)],
                              erows[b], el[b]).wait()

    def s_start(j, b):
        pltpu.async_copy(rows[b], acc_n.at[idx_d.at[j]], ss[b], add=True)

    def s_drain(b):
        pltpu.make_async_copy(rows[b], acc_n.at[idx_d.at[0]], ss[b]).wait()

    def es_start(j, b):
        pltpu.async_copy(erows[b], acc_e.at[idx_d.at[j]], es[b], add=True)

    def es_drain(b):
        pltpu.make_async_copy(erows[b], acc_e.at[idx_d.at[0]],
                              es[b]).wait()

    isl_start(0, 0)
    isl_start(1, 1)
    isl_wait(0)
    for b in range(3):
        g_start(isr[0].at[b], b)

    @pl.when(on_c0)
    def _():
        for b in range(3):
            e_start(b, b)

    @pl.when(on_c1)
    def _():
        zero16 = jnp.zeros((16,), jnp.float32)

        @pl.loop(0, N // 16)
        def _z(g):
            deg_l[pl.ds(g * 16, 16)] = zero16

        ones16 = jnp.ones((16,), jnp.float32)

        @pl.loop(0, NCHA)
        def _hist(r):
            for k in range(CHA // 16):
                d = idx_d[r, pl.ds(k * 16, 16)]
                plsc.addupdate_scatter(deg_l, [d], ones16)

    @pl.loop(0, NGRP // 2)
    def _dgrp(G):
        for e in range(2):
            g = G * 2 + e
            if e == 0:
                isl_wait(1)
            else:
                @pl.when(G < NGRP // 2 - 1)
                def _():
                    isl_wait(0)
            for b in range(5):
                j = g * 5 + b
                g_wait(b)
                s_start(j, b)

                @pl.when(on_c0)
                def _():
                    e_wait(b)
                    es_start(j, b)

                b3 = (b + 3) % 5
                @pl.when((j >= 2) & (j + 3 < NCHA))
                def _():
                    s_drain(b3)

                @pl.when(on_c0 & (j >= 2) & (j + 3 < NCHA))
                def _():
                    es_drain(b3)

                @pl.when(j + 3 < NCHA)
                def _():
                    if b < 2:
                        g_start(isr[e].at[b + 3], b3)
                    else:
                        g_start(isr[1 - e].at[b - 2], b3)

                @pl.when(on_c0 & (j + 3 < NCHA))
                def _():
                    e_start(j + 3, b3)
            @pl.when(g + 2 < NGRP)
            def _():
                isl_start(g + 2, e)

    for b in range(5):
        s_drain(b)

        @pl.when(on_c0)
        def _():
            es_drain(b)

    plsc.subcore_barrier()

    pltpu.sync_copy(acc_n.at[pl.ds(r0, RPT_A)],
                    out_n.at[cid].at[pl.ds(r0, RPT_A)])

    @pl.when(on_c0)
    def _():
        pltpu.sync_copy(acc_e.at[pl.ds(r0, RPT_A)],
                        out_e.at[pl.ds(r0, RPT_A)])

    @pl.when(on_c1)
    def _():
        pltpu.sync_copy(deg_l, out_deg.at[sid])


BLK = 1000


def _dense_body(snp, sep, degp, nf, wmsg, bmsg, wapp, bapp, nh_out, hh_out):
    snl = snp[0]
    snr = snp[1]
    se = sep[...]
    deg = jnp.sum(degp[...], axis=1)[:, None]
    inv = 1.0 / jnp.maximum(deg, 1.0)
    w1l = wmsg[:DH, :]
    w1r = wmsg[DH:DIN, :]
    w2 = wmsg[DIN:, :]
    msum = (jnp.dot(snl * inv, w1l, preferred_element_type=jnp.float32)
            + jnp.dot(snr * inv, w1r, preferred_element_type=jnp.float32)
            + jnp.dot(se * inv, w2, preferred_element_type=jnp.float32))
    hn = msum + jnp.where(deg > 0.0, 1.0, 0.0) * bmsg[0]
    wa1 = wapp[:DIN, :]
    wa2 = wapp[DIN:, :]
    pre = (jnp.dot(nf[0], wa1, preferred_element_type=jnp.float32)
           + jnp.dot(hn, wa2, preferred_element_type=jnp.float32)
           + bapp[0])
    nh = jnp.maximum(pre, 0.0)
    nh_out[0] = nh
    hh_out[0] = 0.5 * nh


def _dense(snp, sep, degt, nfeats3, W_msg, b_msg, W_apply, b_apply):
    grid = (N // BLK,)
    return pl.pallas_call(
        _dense_body,
        grid=grid,
        in_specs=[
            pl.BlockSpec((NC, BLK, DH), lambda i: (0, i, 0)),
            pl.BlockSpec((BLK, DE), lambda i: (i, 0)),
            pl.BlockSpec((BLK, NS), lambda i: (i, 0)),
            pl.BlockSpec((1, BLK, DIN), lambda i: (0, i, 0)),
            pl.BlockSpec((DIN + DE, DOUT), lambda i: (0, 0)),
            pl.BlockSpec((1, DOUT), lambda i: (0, 0)),
            pl.BlockSpec((DIN + DOUT, DOUT), lambda i: (0, 0)),
            pl.BlockSpec((1, DOUT), lambda i: (0, 0)),
        ],
        out_specs=[
            pl.BlockSpec((1, BLK, DOUT), lambda i: (0, i, 0)),
            pl.BlockSpec((1, BLK, DOUT), lambda i: (0, i, 0)),
        ],
        out_shape=[
            jax.ShapeDtypeStruct((1, N, DOUT), jnp.float32),
            jax.ShapeDtypeStruct((1, N, DOUT), jnp.float32),
        ],
    )(snp, sep, degt, nfeats3, W_msg, b_msg, W_apply, b_apply)


@functools.partial(
    pl.kernel,
    out_type=jax.ShapeDtypeStruct((E, DOUT), jnp.float32),
    mesh=_mesh,
    scratch_types=[
        pltpu.VMEM((NCHC, CHC), jnp.int32),
        pltpu.VMEM((NCHC, CHC), jnp.int32),
        [pltpu.VMEM((CHC, DOUT), jnp.float32) for _ in range(4)],
        pltpu.VMEM_SHARED((N, DOUT), jnp.float32),
        [pltpu.SemaphoreType.DMA for _ in range(4)],
        [pltpu.SemaphoreType.DMA for _ in range(4)],
        [pltpu.SemaphoreType.DMA for _ in range(4)],
    ],
    compiler_params=_sc_params,
)
def _edge_kernel(hh, src_r, dst_r, out,
                 idx_s, idx_d, bufo, hh_s, ga, gb, os):
    cid = lax.axis_index("c")
    sid = lax.axis_index("s")
    tid = cid * NS + sid

    pltpu.sync_copy(hh.at[pl.ds(sid * NPS, NPS)],
                    hh_s.at[pl.ds(sid * NPS, NPS)])

    pltpu.sync_copy(src_r.at[tid], idx_s)
    pltpu.sync_copy(dst_r.at[tid], idx_d)

    plsc.subcore_barrier()

    ebase = tid * EPT

    def ga_start(j, b):
        pltpu.async_copy(hh.at[idx_s.at[j]], bufo[b], ga[b])

    def ga_wait(b):
        pltpu.make_async_copy(hh.at[idx_s.at[0]], bufo[b], ga[b]).wait()

    def gb_start(j, b):
        pltpu.async_copy(hh_s.at[idx_d.at[j]], bufo[b], gb[b], add=True)

    def gb_wait(b):
        pltpu.make_async_copy(hh_s.at[idx_d.at[0]], bufo[b], gb[b]).wait()

    def o_start(j, b):
        pltpu.async_copy(bufo[b], out.at[pl.ds(ebase + j * CHC, CHC)],
                         os[b])

    def o_wait(b):
        pltpu.make_async_copy(bufo[b],
                              out.at[pl.ds(ebase, CHC)], os[b]).wait()

    ga_start(0, 0)
    ga_start(1, 1)

    @pl.loop(0, NCHC // 4)
    def _grp(G):
        for b in range(4):
            j = G * 4 + b
            ga_wait(b)
            gb_start(j, b)
            gb_wait(b)
            o_start(j, b)
            b2 = (b + 2) % 4

            @pl.when(j + 2 < NCHC)
            def _():
                @pl.when(j >= 2)
                def _():
                    o_wait(b2)

                ga_start(j + 2, b2)

    for k in range(4):
        o_wait((NCHC - 4 + k) % 4)


@jax.jit
def kernel(nfeats, efeats, edge_index, W_msg, b_msg, W_apply, b_apply):
    nfeats2 = nfeats.reshape(N, DIN)
    efeats2 = efeats.reshape(E, DE)
    src = edge_index[0].astype(jnp.int32)
    dst = edge_index[1].astype(jnp.int32)

    nf_view = nfeats2.reshape(2 * N, DH)

    zn = jnp.zeros((NACC, DH), jnp.float32)
    ze = jnp.zeros((NACC, DE), jnp.float32)

    snp, sep, degp = _scatter_kernel(
        nf_view, efeats2,
        src.reshape(NS, NCHA, CHA), dst.reshape(NS, NCHA, CHA), zn, ze)

    nh3, hh3 = _dense(snp, sep, degp.T, nfeats2[None], W_msg,
                      b_msg[None], W_apply, b_apply[None])
    new_h = nh3.reshape(N, 1, DOUT)
    hh = hh3.reshape(N, DOUT)

    e_out = _edge_kernel(hh, src.reshape(NW, NCHC, CHC),
                         dst.reshape(NW, NCHC, CHC))
    return new_h, e_out.reshape(E, 1, DOUT)


# speedup vs baseline: 1.0774x; 1.0774x over previous
import functools

import jax
import jax.numpy as jnp
from jax import lax
from jax.experimental import pallas as pl
from jax.experimental.pallas import tpu as pltpu
from jax.experimental.pallas import tpu_sc as plsc

N = 10000
E = 320000
DIN = 128
DH = DIN // 2
DE = 16
DOUT = 128

NC = 2
NS = 16
NW = NC * NS
NACC = 10240
RPT_A = NACC // NS

EPS = E // NS
CHA = 80
NCHA = EPS // CHA
NGRP = NCHA // 5

EPT = E // NW
CHC = 50
NCHC = EPT // CHC
NPS = N // NS

_mesh = plsc.VectorSubcoreMesh(
    core_axis_name="c", subcore_axis_name="s", num_cores=NC, num_subcores=NS)
_sc_params = pltpu.CompilerParams(
    use_tc_tiling_on_sc=False, needs_layout_passes=False)


@functools.partial(
    pl.kernel,
    out_type=(
        jax.ShapeDtypeStruct((NC, NACC, DH), jnp.float32),
        jax.ShapeDtypeStruct((NACC, DE), jnp.float32),
        jax.ShapeDtypeStruct((NS, N), jnp.float32),
    ),
    mesh=_mesh,
    scratch_types=[
        [pltpu.VMEM((5, CHA), jnp.int32) for _ in range(2)],
        pltpu.VMEM((NCHA, CHA), jnp.int32),
        [pltpu.VMEM((CHA, DH), jnp.float32) for _ in range(5)],
        [pltpu.VMEM((CHA, DE), jnp.float32) for _ in range(5)],
        pltpu.VMEM((N,), jnp.float32),
        pltpu.VMEM_SHARED((NACC, DH), jnp.float32),
        pltpu.VMEM_SHARED((NACC, DE), jnp.float32),
        [pltpu.SemaphoreType.DMA for _ in range(2)],
        [pltpu.SemaphoreType.DMA for _ in range(5)],
        [pltpu.SemaphoreType.DMA for _ in range(5)],
        [pltpu.SemaphoreType.DMA for _ in range(5)],
        [pltpu.SemaphoreType.DMA for _ in range(5)],
    ],
    compiler_params=_sc_params,
)
def _scatter_kernel(nf_view, efeats2, src_r, dst_r, zn, ze,
                    out_n, out_e, out_deg,
                    isr, idx_d, rows, erows, deg_l,
                    acc_n, acc_e, isl, gs, ss, el, es):
    cid = lax.axis_index("c")
    sid = lax.axis_index("s")
    on_c0 = cid == 0
    on_c1 = cid == 1
    cvec = jnp.zeros((16,), jnp.int32) + cid
    two = jnp.full((16,), 2, jnp.int32)

    r0 = sid * RPT_A
    pltpu.async_copy(zn.at[pl.ds(r0, RPT_A)], acc_n.at[pl.ds(r0, RPT_A)],
                     gs[4])
    pltpu.async_copy(dst_r.at[sid], idx_d, ss[4])

    @pl.when(on_c0)
    def _():
        pltpu.async_copy(ze.at[pl.ds(r0, RPT_A)],
                         acc_e.at[pl.ds(r0, RPT_A)], el[4])

    pltpu.make_async_copy(zn.at[pl.ds(r0, RPT_A)],
                          acc_n.at[pl.ds(r0, RPT_A)], gs[4]).wait()
    pltpu.make_async_copy(dst_r.at[sid], idx_d, ss[4]).wait()

    @pl.when(on_c0)
    def _():
        pltpu.make_async_copy(ze.at[pl.ds(r0, RPT_A)],
                              acc_e.at[pl.ds(r0, RPT_A)], el[4]).wait()

    ebase = sid * EPS

    def isl_start(g, e):
        pltpu.async_copy(src_r.at[sid].at[pl.ds(g * 5, 5)], isr[e], isl[e])

    def isl_wait(e):
        pltpu.make_async_copy(src_r.at[sid].at[pl.ds(0, 5)],
                              isr[e], isl[e]).wait()
        for r in range(5):
            for k in range(CHA // 16):
                sl = pl.ds(k * 16, 16)
                isr[e][r, sl] = isr[e][r, sl] * two + cvec

    def g_start(row_ref, b):
        pltpu.async_copy(nf_view.at[row_ref], rows[b], gs[b])

    def g_wait(b):
        pltpu.make_async_copy(nf_view.at[isr[0].at[0]], rows[b],
                              gs[b]).wait()

    def e_start(j, b):
        pltpu.async_copy(efeats2.at[pl.ds(ebase + j * CHA, CHA)],
                         erows[b], el[b])

    def e_wait(b):
        pltpu.make_async_copy(efeats2.at[pl.ds(0, CHA)],
                              erows[b], el[b]).wait()

    def s_start(j, b):
        pltpu.async_copy(rows[b], acc_n.at[idx_d.at[j]], ss[b], add=True)

    def s_drain(b):
        pltpu.make_async_copy(rows[b], acc_n.at[idx_d.at[0]], ss[b]).wait()

    def es_start(j, b):
        pltpu.async_copy(erows[b], acc_e.at[idx_d.at[j]], es[b], add=True)

    def es_drain(b):
        pltpu.make_async_copy(erows[b], acc_e.at[idx_d.at[0]],
                              es[b]).wait()

    isl_start(0, 0)
    isl_start(1, 1)
    isl_wait(0)
    for b in range(3):
        g_start(isr[0].at[b], b)

    @pl.when(on_c0)
    def _():
        for b in range(3):
            e_start(b, b)

    @pl.when(on_c1)
    def _():
        zero16 = jnp.zeros((16,), jnp.float32)

        @pl.loop(0, N // 16)
        def _z(g):
            deg_l[pl.ds(g * 16, 16)] = zero16

        ones16 = jnp.ones((16,), jnp.float32)

        @pl.loop(0, NCHA)
        def _hist(r):
            for k in range(CHA // 16):
                d = idx_d[r, pl.ds(k * 16, 16)]
                plsc.addupdate_scatter(deg_l, [d], ones16)

    @pl.loop(0, NGRP // 2)
    def _dgrp(G):
        for e in range(2):
            g = G * 2 + e
            if e == 0:
                isl_wait(1)
            else:
                @pl.when(G < NGRP // 2 - 1)
                def _():
                    isl_wait(0)
            for b in range(5):
                j = g * 5 + b
                g_wait(b)
                s_start(j, b)

                @pl.when(on_c0)
                def _():
                    e_wait(b)
                    es_start(j, b)

                b3 = (b + 3) % 5
                @pl.when((j >= 2) & (j + 3 < NCHA))
                def _():
                    s_drain(b3)

                @pl.when(on_c0 & (j >= 2) & (j + 3 < NCHA))
                def _():
                    es_drain(b3)

                @pl.when(j + 3 < NCHA)
                def _():
                    if b < 2:
                        g_start(isr[e].at[b + 3], b3)
                    else:
                        g_start(isr[1 - e].at[b - 2], b3)

                @pl.when(on_c0 & (j + 3 < NCHA))
                def _():
                    e_start(j + 3, b3)
            @pl.when(g + 2 < NGRP)
            def _():
                isl_start(g + 2, e)

    for b in range(5):
        s_drain(b)

        @pl.when(on_c0)
        def _():
            es_drain(b)

    plsc.subcore_barrier()

    pltpu.sync_copy(acc_n.at[pl.ds(r0, RPT_A)],
                    out_n.at[cid].at[pl.ds(r0, RPT_A)])

    @pl.when(on_c0)
    def _():
        pltpu.sync_copy(acc_e.at[pl.ds(r0, RPT_A)],
                        out_e.at[pl.ds(r0, RPT_A)])

    @pl.when(on_c1)
    def _():
        pltpu.sync_copy(deg_l, out_deg.at[sid])


BLK = 1000


def _dense_body(snp, sep, degp, nf, wmsg, bmsg, wapp, bapp, nh_out, hh_out):
    snl = snp[0]
    snr = snp[1]
    se = sep[...]
    deg = jnp.sum(degp[...], axis=1)[:, None]
    inv = 1.0 / jnp.maximum(deg, 1.0)
    w1l = wmsg[:DH, :]
    w1r = wmsg[DH:DIN, :]
    w2 = wmsg[DIN:, :]
    msum = (jnp.dot(snl * inv, w1l, preferred_element_type=jnp.float32)
            + jnp.dot(snr * inv, w1r, preferred_element_type=jnp.float32)
            + jnp.dot(se * inv, w2, preferred_element_type=jnp.float32))
    hn = msum + jnp.where(deg > 0.0, 1.0, 0.0) * bmsg[0]
    wa1 = wapp[:DIN, :]
    wa2 = wapp[DIN:, :]
    pre = (jnp.dot(nf[0], wa1, preferred_element_type=jnp.float32)
           + jnp.dot(hn, wa2, preferred_element_type=jnp.float32)
           + bapp[0])
    nh = jnp.maximum(pre, 0.0)
    nh_out[0] = nh
    hh_out[0] = 0.5 * nh


def _dense(snp, sep, degt, nfeats3, W_msg, b_msg, W_apply, b_apply):
    grid = (N // BLK,)
    return pl.pallas_call(
        _dense_body,
        grid=grid,
        in_specs=[
            pl.BlockSpec((NC, BLK, DH), lambda i: (0, i, 0)),
            pl.BlockSpec((BLK, DE), lambda i: (i, 0)),
            pl.BlockSpec((BLK, NS), lambda i: (i, 0)),
            pl.BlockSpec((1, BLK, DIN), lambda i: (0, i, 0)),
            pl.BlockSpec((DIN + DE, DOUT), lambda i: (0, 0)),
            pl.BlockSpec((1, DOUT), lambda i: (0, 0)),
            pl.BlockSpec((DIN + DOUT, DOUT), lambda i: (0, 0)),
            pl.BlockSpec((1, DOUT), lambda i: (0, 0)),
        ],
        out_specs=[
            pl.BlockSpec((1, BLK, DOUT), lambda i: (0, i, 0)),
            pl.BlockSpec((1, BLK, DOUT), lambda i: (0, i, 0)),
        ],
        out_shape=[
            jax.ShapeDtypeStruct((1, N, DOUT), jnp.float32),
            jax.ShapeDtypeStruct((1, N, DOUT), jnp.float32),
        ],
    )(snp, sep, degt, nfeats3, W_msg, b_msg, W_apply, b_apply)


@functools.partial(
    pl.kernel,
    out_type=jax.ShapeDtypeStruct((E, DOUT), jnp.float32),
    mesh=_mesh,
    scratch_types=[
        pltpu.VMEM((NCHC, CHC), jnp.int32),
        pltpu.VMEM((NCHC, CHC), jnp.int32),
        [pltpu.VMEM((CHC, DOUT), jnp.float32) for _ in range(4)],
        pltpu.VMEM_SHARED((N, DOUT), jnp.float32),
        [pltpu.SemaphoreType.DMA for _ in range(4)],
        [pltpu.SemaphoreType.DMA for _ in range(4)],
        [pltpu.SemaphoreType.DMA for _ in range(4)],
    ],
    compiler_params=_sc_params,
)
def _edge_kernel(hh, src_r, dst_r, out,
                 idx_s, idx_d, bufo, hh_s, ga, gb, os):
    cid = lax.axis_index("c")
    sid = lax.axis_index("s")
    tid = cid * NS + sid

    pltpu.async_copy(hh.at[pl.ds(sid * NPS, NPS)],
                     hh_s.at[pl.ds(sid * NPS, NPS)], ga[0])
    pltpu.async_copy(src_r.at[tid], idx_s, gb[0])
    pltpu.async_copy(dst_r.at[tid], idx_d, os[0])
    pltpu.make_async_copy(hh.at[pl.ds(sid * NPS, NPS)],
                          hh_s.at[pl.ds(sid * NPS, NPS)], ga[0]).wait()
    pltpu.make_async_copy(src_r.at[tid], idx_s, gb[0]).wait()
    pltpu.make_async_copy(dst_r.at[tid], idx_d, os[0]).wait()

    plsc.subcore_barrier()

    ebase = tid * EPT

    def ga_start(j, b):
        pltpu.async_copy(hh_s.at[idx_s.at[j]], bufo[b], ga[b])

    def ga_wait(b):
        pltpu.make_async_copy(hh_s.at[idx_s.at[0]], bufo[b], ga[b]).wait()

    def gb_start(j, b):
        pltpu.async_copy(hh_s.at[idx_d.at[j]], bufo[b], gb[b], add=True)

    def gb_wait(b):
        pltpu.make_async_copy(hh_s.at[idx_d.at[0]], bufo[b], gb[b]).wait()

    def o_start(j, b):
        pltpu.async_copy(bufo[b], out.at[pl.ds(ebase + j * CHC, CHC)],
                         os[b])

    def o_wait(b):
        pltpu.make_async_copy(bufo[b],
                              out.at[pl.ds(ebase, CHC)], os[b]).wait()

    ga_start(0, 0)
    ga_start(1, 1)

    @pl.loop(0, NCHC // 4)
    def _grp(G):
        for b in range(4):
            j = G * 4 + b
            ga_wait(b)
            gb_start(j, b)
            gb_wait(b)
            o_start(j, b)
            b2 = (b + 2) % 4

            @pl.when(j + 2 < NCHC)
            def _():
                @pl.when(j >= 2)
                def _():
                    o_wait(b2)

                ga_start(j + 2, b2)

    for k in range(4):
        o_wait((NCHC - 4 + k) % 4)


@jax.jit
def kernel(nfeats, efeats, edge_index, W_msg, b_msg, W_apply, b_apply):
    nfeats2 = nfeats.reshape(N, DIN)
    efeats2 = efeats.reshape(E, DE)
    src = edge_index[0].astype(jnp.int32)
    dst = edge_index[1].astype(jnp.int32)

    nf_view = nfeats2.reshape(2 * N, DH)

    zn = jnp.zeros((NACC, DH), jnp.float32)
    ze = jnp.zeros((NACC, DE), jnp.float32)

    snp, sep, degp = _scatter_kernel(
        nf_view, efeats2,
        src.reshape(NS, NCHA, CHA), dst.reshape(NS, NCHA, CHA), zn, ze)

    nh3, hh3 = _dense(snp, sep, degp.T, nfeats2[None], W_msg,
                      b_msg[None], W_apply, b_apply[None])
    new_h = nh3.reshape(N, 1, DOUT)
    hh = hh3.reshape(N, DOUT)

    e_out = _edge_kernel(hh, src.reshape(NW, NCHC, CHC),
                         dst.reshape(NW, NCHC, CHC))
    return new_h, e_out.reshape(E, 1, DOUT)
